# Initial kernel scaffold; baseline (speedup 1.0000x reference)
import jax
import jax.numpy as jnp
from jax.experimental import pallas as pl


def kernel(x, edge_index, Wl0, bl0, Wr0, Wl1, bl1, Wr1, Wl2, bl2, Wr2, Wl3, bl3, Wr3, Wl4, bl4, Wr4):
    raise NotImplementedError("write your pallas kernel here")


# speedup vs baseline: 40.6294x; 40.6294x over previous
import functools

import jax
import jax.numpy as jnp
from jax import lax
from jax.experimental import pallas as pl
from jax.experimental.pallas import tpu as pltpu
from jax.experimental.pallas import tpu_sc as plsc

NC = 2
NS = 16
NW = NC * NS
CH = 128
DH = 32


def _sc_agg_kernel(np_rows, n_chunk_rows):
    Np = np_rows
    R = n_chunk_rows
    PR = Np // NS

    mesh = plsc.VectorSubcoreMesh(core_axis_name="c", subcore_axis_name="s")

    @functools.partial(
        pl.kernel,
        out_type=jax.ShapeDtypeStruct((NC * Np, DH), jnp.float32),
        mesh=mesh,
        scratch_types=[
            pltpu.VMEM((R, CH), jnp.int32),
            pltpu.VMEM((R, CH), jnp.int32),
            pltpu.VMEM((CH, DH), jnp.float32),
            pltpu.VMEM((CH, DH), jnp.float32),
            pltpu.VMEM((CH, DH), jnp.float32),
            pltpu.VMEM((CH, DH), jnp.float32),
            pltpu.VMEM_SHARED((Np, DH), jnp.float32),
            pltpu.SemaphoreType.DMA,
            pltpu.SemaphoreType.DMA,
            pltpu.SemaphoreType.DMA,
            pltpu.SemaphoreType.DMA,
        ],
        compiler_params=pltpu.CompilerParams(use_tc_tiling_on_sc=False),
    )
    def agg(y_hbm, srcr_hbm, dstr_hbm, acc_hbm, sidx, didx,
            rows0, rows1, rows2, rows3, acc_sp, gs0, gs1, gs2, gs3):
        rows = [rows0, rows1, rows2, rows3]
        gs = [gs0, gs1, gs2, gs3]
        nbuf = 4
        c = lax.axis_index("c")
        s = lax.axis_index("s")
        wid = s * NC + c

        zero16 = jnp.zeros((16,), jnp.float32)

        def zbody(j, _):
            r = j // 2
            col = (j % 2) * 16
            rows0[r, pl.ds(col, 16)] = zero16
            return 0

        lax.fori_loop(0, CH * 2, zbody, 0)
        for t in range(PR // CH):
            pltpu.sync_copy(rows0, acc_sp.at[pl.ds(s * PR + t * CH, CH)])
        plsc.subcore_barrier()

        pltpu.sync_copy(srcr_hbm.at[pl.ds(wid * R, R)], sidx)
        pltpu.sync_copy(dstr_hbm.at[pl.ds(wid * R, R)], didx)

        for b in range(nbuf):
            pltpu.async_copy(y_hbm.at[sidx.at[b]], rows[b], gs[b])
        nj = R // nbuf

        def body(j, _):
            for b in range(nbuf):
                k = j * nbuf + b
                pltpu.make_async_copy(y_hbm.at[sidx.at[k]], rows[b],
                                      gs[b]).wait()
                pltpu.sync_copy(rows[b], acc_sp.at[didx.at[k]], add=True)

                @pl.when(j < nj - 1)
                def _():
                    pltpu.async_copy(y_hbm.at[sidx.at[k + nbuf]], rows[b],
                                     gs[b])
            return 0

        lax.fori_loop(0, nj, body, 0)
        plsc.subcore_barrier()

        pltpu.sync_copy(acc_sp.at[pl.ds(s * PR, PR)],
                        acc_hbm.at[pl.ds(c * Np + s * PR, PR)])

    return agg


def _sc_cnt_kernel(np_rows, n_chunk_rows):
    Np = np_rows
    R = n_chunk_rows
    PR = Np // NS

    mesh = plsc.VectorSubcoreMesh(core_axis_name="c", subcore_axis_name="s")

    @functools.partial(
        pl.kernel,
        out_type=jax.ShapeDtypeStruct((NC * Np,), jnp.float32),
        mesh=mesh,
        scratch_types=[
            pltpu.VMEM((R, CH), jnp.int32),
            pltpu.VMEM((CH,), jnp.float32),
            pltpu.VMEM_SHARED((Np,), jnp.float32),
        ],
        compiler_params=pltpu.CompilerParams(use_tc_tiling_on_sc=False),
    )
    def cnt(dstr_hbm, cnt_hbm, didx, ones_v, cnt_sp):
        c = lax.axis_index("c")
        s = lax.axis_index("s")
        wid = s * NC + c

        zero16 = jnp.zeros((16,), jnp.float32)
        for j in range(CH // 16):
            ones_v[pl.ds(j * 16, 16)] = zero16
        for t in range(PR // CH):
            pltpu.sync_copy(ones_v, cnt_sp.at[pl.ds(s * PR + t * CH, CH)])
        plsc.subcore_barrier()

        one16 = jnp.ones((16,), jnp.float32)
        for j in range(CH // 16):
            ones_v[pl.ds(j * 16, 16)] = one16

        pltpu.sync_copy(dstr_hbm.at[pl.ds(wid * R, R)], didx)

        def body(k, _):
            pltpu.sync_copy(ones_v, cnt_sp.at[didx.at[k]], add=True)
            return 0

        lax.fori_loop(0, R, body, 0)
        plsc.subcore_barrier()
        pltpu.sync_copy(cnt_sp.at[pl.ds(s * PR, PR)],
                        cnt_hbm.at[pl.ds(c * Np + s * PR, PR)])

    return cnt


def _mm_t(x, w):
    return lax.dot_general(x, w, dimension_numbers=(((1,), (1,)), ((), ())),
                           preferred_element_type=jnp.float32)


def _tc_transform0(x_ref, wl_ref, wr_ref, y_ref, r_ref):
    xv = x_ref[...]
    y_ref[...] = _mm_t(xv, wl_ref[...])
    r_ref[...] = _mm_t(xv, wr_ref[...])


def _tc_combine_next(skip):
    def body(acc_ref, cnt_ref, r_ref, x_ref, bl_ref, wln_ref, wrn_ref,
             xn_ref, yn_ref, rn_ref):
        npr = acc_ref.shape[0] // 2
        acc = acc_ref[0:npr, :] + acc_ref[npr:2 * npr, :]
        cnt = cnt_ref[0:npr, :] + cnt_ref[npr:2 * npr, :]
        inv = 1.0 / jnp.maximum(cnt, 1.0)
        h = acc * inv + bl_ref[...] + r_ref[...]
        if skip:
            h = h + x_ref[...]
        xn = jnp.maximum(h, 0.0)
        xn_ref[...] = xn
        yn_ref[...] = _mm_t(xn, wln_ref[...])
        rn_ref[...] = _mm_t(xn, wrn_ref[...])
    return body


def _tc_combine_final(acc_ref, cnt_ref, r_ref, bl_ref, out_ref):
    npr = acc_ref.shape[0] // 2
    acc = acc_ref[0:npr, :] + acc_ref[npr:2 * npr, :]
    cnt = cnt_ref[0:npr, :] + cnt_ref[npr:2 * npr, :]
    inv = 1.0 / jnp.maximum(cnt, 1.0)
    out_ref[...] = acc * inv + bl_ref[...] + r_ref[...]


def kernel(x, edge_index, Wl0, bl0, Wr0, Wl1, bl1, Wr1, Wl2, bl2, Wr2,
           Wl3, bl3, Wr3, Wl4, bl4, Wr4):
    N, d_in = x.shape
    E = edge_index.shape[1]
    params = [(Wl0, bl0, Wr0), (Wl1, bl1, Wr1), (Wl2, bl2, Wr2),
              (Wl3, bl3, Wr3), (Wl4, bl4, Wr4)]

    Np = (N + 1 + NS * CH - 1) // (NS * CH) * (NS * CH)
    R = (E + NW * CH - 1) // (NW * CH)
    R = (R + 7) // 8 * 8
    Ep = NW * R * CH

    src = edge_index[0]
    dst = edge_index[1]
    pad = Ep - E
    pad_ar = jnp.arange(pad, dtype=jnp.int32)
    src_p = jnp.concatenate([src, pad_ar % N])
    dst_p = jnp.concatenate([dst, N + pad_ar % (Np - N)])
    srcr = src_p.reshape(NW * R, CH)
    dstr = dst_p.reshape(NW * R, CH)

    xp = jnp.pad(x, ((0, Np - N), (0, 0)))

    agg_call = _sc_agg_kernel(Np, R)
    cnt_call = _sc_cnt_kernel(Np, R)

    cntP = cnt_call(dstr)
    cnt2 = cntP.reshape(NC * Np, 1)

    y, r = pl.pallas_call(
        _tc_transform0,
        out_shape=(jax.ShapeDtypeStruct((Np, DH), jnp.float32),
                   jax.ShapeDtypeStruct((Np, DH), jnp.float32)),
    )(xp, Wl0, Wr0)

    xi = xp
    for i in range(4):
        accP = agg_call(y, srcr, dstr)
        Wln, bln, Wrn = params[i + 1]
        xi, y, r = pl.pallas_call(
            _tc_combine_next(skip=(i > 0)),
            out_shape=(jax.ShapeDtypeStruct((Np, DH), jnp.float32),
                       jax.ShapeDtypeStruct((Np, DH), jnp.float32),
                       jax.ShapeDtypeStruct((Np, DH), jnp.float32)),
        )(accP, cnt2, r, xi, params[i][1].reshape(1, DH), Wln, Wrn)

    accP = agg_call(y, srcr, dstr)
    out = pl.pallas_call(
        _tc_combine_final,
        out_shape=jax.ShapeDtypeStruct((Np, DH), jnp.float32),
    )(accP, cnt2, r, bl4.reshape(1, DH))
    return out[:N]
